# baseline (speedup 1.0000x reference)
import functools

import jax
import jax.numpy as jnp
from jax import lax
from jax.experimental import pallas as pl
from jax.experimental.pallas import tpu as pltpu
from jax.experimental.pallas import tpu_sc as plsc

NUM_PATCHES = 8192
PROJ_DIM = 1024


@functools.lru_cache(maxsize=None)
def _make_copy_kernel():
    info = plsc.get_sparse_core_info()
    nw = info.num_cores * info.num_subcores
    rows_per_w = NUM_PATCHES // nw

    mesh = plsc.VectorSubcoreMesh(core_axis_name="c", subcore_axis_name="s")

    @functools.partial(
        pl.kernel,
        out_type=jax.ShapeDtypeStruct((NUM_PATCHES, PROJ_DIM), jnp.float32),
        mesh=mesh,
        scratch_types=[pltpu.SemaphoreType.DMA],
    )
    def copy_rows(table_hbm, out_hbm, sem):
        wid = lax.axis_index("s") * info.num_cores + lax.axis_index("c")
        base = wid * rows_per_w
        pltpu.async_copy(
            table_hbm.at[pl.ds(base, rows_per_w)],
            out_hbm.at[pl.ds(base, rows_per_w)],
            sem,
        ).wait()

    return copy_rows


def kernel(tokens, table):
    del tokens
    out = _make_copy_kernel()(table)
    return out[None]


# speedup vs baseline: 24.2655x; 24.2655x over previous
import functools

import jax
import jax.numpy as jnp
from jax import lax
from jax.experimental import pallas as pl
from jax.experimental.pallas import tpu as pltpu
from jax.experimental.pallas import tpu_sc as plsc

NUM_PATCHES = 8192
PROJ_DIM = 1024


CHUNK_ROWS = 32


@functools.lru_cache(maxsize=None)
def _make_copy_kernel():
    info = plsc.get_sparse_core_info()
    nw = info.num_cores * info.num_subcores
    rows_per_w = NUM_PATCHES // nw
    n_ch = rows_per_w // CHUNK_ROWS

    mesh = plsc.VectorSubcoreMesh(core_axis_name="c", subcore_axis_name="s")

    @functools.partial(
        pl.kernel,
        out_type=jax.ShapeDtypeStruct((NUM_PATCHES, PROJ_DIM), jnp.float32),
        mesh=mesh,
        scratch_types=[
            pltpu.VMEM((CHUNK_ROWS, PROJ_DIM), jnp.float32),
            pltpu.VMEM((CHUNK_ROWS, PROJ_DIM), jnp.float32),
            pltpu.SemaphoreType.DMA,
            pltpu.SemaphoreType.DMA,
            pltpu.SemaphoreType.DMA,
            pltpu.SemaphoreType.DMA,
        ],
    )
    def copy_rows(table_hbm, out_hbm, buf0, buf1, si0, si1, so0, so1):
        wid = lax.axis_index("s") * info.num_cores + lax.axis_index("c")
        base = wid * rows_per_w
        bufs = (buf0, buf1)
        sin = (si0, si1)
        sout = (so0, so1)

        def in_copy(i):
            b = i % 2
            return pltpu.async_copy(
                table_hbm.at[pl.ds(base + i * CHUNK_ROWS, CHUNK_ROWS)],
                bufs[b], sin[b])

        def out_copy(i):
            b = i % 2
            return pltpu.async_copy(
                bufs[b],
                out_hbm.at[pl.ds(base + i * CHUNK_ROWS, CHUNK_ROWS)],
                sout[b])

        h_in = [None] * n_ch
        h_out = [None] * n_ch
        h_in[0] = in_copy(0)
        for i in range(n_ch):
            if i + 1 < n_ch:
                if i >= 1:
                    h_out[i - 1].wait()
                h_in[i + 1] = in_copy(i + 1)
            h_in[i].wait()
            h_out[i] = out_copy(i)
        if n_ch >= 2:
            h_out[n_ch - 2].wait()
        h_out[n_ch - 1].wait()

    return copy_rows


def kernel(tokens, table):
    del tokens
    out = _make_copy_kernel()(table)
    return out[None]
